# baseline (speedup 1.0000x reference)
import functools

import jax
import jax.numpy as jnp
from jax import lax
from jax.experimental import pallas as pl
from jax.experimental.pallas import tpu as pltpu
from jax.experimental.pallas import tpu_sc as plsc

_NUM_SENTENCES = 100000
_SEQ = 20
_DIM = 64

_NC = 2
_NS = 16
_NW = _NC * _NS
_LANES = 16

_BS = 3584
_GI = 128
_PDIM = 128


def _pool_body(ct_ref, pooled_ref):
    x = ct_ref[...]
    s = jnp.sum(x, axis=0) * (1.0 / _SEQ)
    t = jnp.transpose(s, (1, 0))
    pooled_ref[...] = jnp.concatenate(
        [t, jnp.zeros((t.shape[0], _PDIM - _DIM), jnp.float32)], axis=1)


def _tc_pool(ct):
    nblk = (_NUM_SENTENCES + _BS - 1) // _BS
    return pl.pallas_call(
        _pool_body,
        grid=(nblk,),
        in_specs=[pl.BlockSpec((_SEQ, _DIM, _BS), lambda i: (0, 0, i))],
        out_specs=pl.BlockSpec((_BS, _PDIM), lambda i: (i, 0)),
        out_shape=jax.ShapeDtypeStruct((_NUM_SENTENCES, _PDIM), jnp.float32),
    )(ct)


def kernel(sentence_ids, dataset_ids, cache):
    batch = sentence_ids.shape[0]
    b_per_w = batch // _NW

    ct = cache.transpose(1, 2, 0)
    pooled = _tc_pool(ct)

    mesh = plsc.VectorSubcoreMesh(
        core_axis_name="c", subcore_axis_name="s",
        num_cores=_NC, num_subcores=_NS)

    @functools.partial(
        pl.kernel,
        mesh=mesh,
        out_type=jax.ShapeDtypeStruct((batch, _DIM), jnp.float32),
        scratch_types=[
            pltpu.VMEM((b_per_w,), jnp.int32),
            pltpu.VMEM((b_per_w,), jnp.int32),
            pltpu.VMEM((b_per_w // _GI, _GI), jnp.int32),
            pltpu.VMEM((2, _GI, _PDIM), jnp.float32),
            pltpu.VMEM((_GI, _DIM), jnp.float32),
            pltpu.SemaphoreType.DMA,
            pltpu.SemaphoreType.DMA,
        ],
    )
    def sc_gather(sid_hbm, did_hbm, pooled_hbm, out_hbm,
                  ids_v, dids_v, idx_v, rows_v, out_v, sem0, sem1):
        sems = (sem0, sem1)
        wid = lax.axis_index("s") * _NC + lax.axis_index("c")
        base = wid * b_per_w
        nch = b_per_w // _GI

        pltpu.sync_copy(sid_hbm.at[pl.ds(base, b_per_w)], ids_v)
        pltpu.sync_copy(did_hbm.at[pl.ds(base, b_per_w)], dids_v)
        for j in range(b_per_w // _LANES):
            sl = pl.ds(j * _LANES, _LANES)
            g, h = divmod(j * _LANES, _GI)
            idx_v[g, pl.ds(h, _LANES)] = (
                ids_v[sl] + dids_v[sl] * _NUM_SENTENCES)

        def issue(g):
            pltpu.async_copy(
                pooled_hbm.at[idx_v.at[g]], rows_v.at[g % 2], sems[g % 2])

        issue(0)
        issue(1)
        for g in range(nch):
            buf = g % 2
            pltpu.make_async_copy(
                pooled_hbm.at[pl.ds(0, _GI)], rows_v.at[buf], sems[buf]).wait()

            def compact_body(s, carry, _buf=buf):
                for d in range(_DIM // _LANES):
                    sl = pl.ds(d * _LANES, _LANES)
                    out_v[s, sl] = rows_v[_buf, s, sl]
                return carry

            lax.fori_loop(0, _GI, compact_body, 0, unroll=False)
            pltpu.sync_copy(out_v, out_hbm.at[pl.ds(base + g * _GI, _GI)])
            if g + 2 < nch:
                issue(g + 2)

    return sc_gather(sentence_ids, dataset_ids, pooled)
